# Initial kernel scaffold; baseline (speedup 1.0000x reference)
import jax
import jax.numpy as jnp
from jax.experimental import pallas as pl


def kernel(x, y, centers):
    raise NotImplementedError("write your pallas kernel here")


# speedup vs baseline: 27.3980x; 27.3980x over previous
import jax
import jax.numpy as jnp
from jax.experimental import pallas as pl

N = 2048
D = 256
K = 512
BR = 256
GRID = N // BR


def _kmeans_kernel(x_ref, y_ref, ct_ref, ynew_ref, loss_ref):
    x = x_ref[...]
    ct = ct_ref[...]
    yl = y_ref[...]

    xc = jax.lax.dot_general(
        x, ct,
        dimension_numbers=(((1,), (0,)), ((), ())),
        preferred_element_type=jnp.float32,
        precision=jax.lax.Precision.HIGHEST,
    )
    c2 = jnp.sum(ct * ct, axis=0, keepdims=True)
    s = c2 - 2.0 * xc

    smin = jnp.min(s, axis=1, keepdims=True)
    col = jax.lax.broadcasted_iota(jnp.int32, (BR, K), 1)
    ynew_ref[...] = jnp.min(jnp.where(s == smin, col, K), axis=1, keepdims=True)

    hit = jnp.where(col == yl, s, 0.0)
    part = (jnp.sum(x * x, axis=(0, 1), keepdims=True)
            + jnp.sum(hit, axis=(0, 1), keepdims=True))

    @pl.when(pl.program_id(0) == 0)
    def _init():
        loss_ref[...] = jnp.zeros((1, 1), jnp.float32)

    loss_ref[...] += part


def kernel(x, y, centers):
    y2 = y.reshape(N, 1).astype(jnp.int32)
    ct = centers.T
    ynew2, loss2 = pl.pallas_call(
        _kmeans_kernel,
        grid=(GRID,),
        in_specs=[
            pl.BlockSpec((BR, D), lambda i: (i, 0)),
            pl.BlockSpec((BR, 1), lambda i: (i, 0)),
            pl.BlockSpec((D, K), lambda i: (0, 0)),
        ],
        out_specs=(
            pl.BlockSpec((BR, 1), lambda i: (i, 0)),
            pl.BlockSpec((1, 1), lambda i: (0, 0)),
        ),
        out_shape=(
            jax.ShapeDtypeStruct((N, 1), jnp.int32),
            jax.ShapeDtypeStruct((1, 1), jnp.float32),
        ),
    )(x, y2, ct)
    return (loss2[0, 0], ynew2.reshape(N))


# speedup vs baseline: 30.6069x; 1.1171x over previous
import jax
import jax.numpy as jnp
from jax.experimental import pallas as pl

N = 2048
D = 256
K = 512
BR = 512
GRID = N // BR


def _kmeans_kernel(x_ref, y_ref, ct_ref, ynew_ref, loss_ref):
    x = x_ref[...]
    ct = ct_ref[...]
    yl = y_ref[...]

    xc = jax.lax.dot_general(
        x, ct,
        dimension_numbers=(((1,), (0,)), ((), ())),
        preferred_element_type=jnp.float32,
        precision=jax.lax.Precision.HIGHEST,
    )
    c2 = jnp.sum(ct * ct, axis=0, keepdims=True)
    s = c2 - 2.0 * xc

    smin = jnp.min(s, axis=1, keepdims=True)
    col = jax.lax.broadcasted_iota(jnp.int32, (BR, K), 1)
    ynew_ref[...] = jnp.min(jnp.where(s == smin, col, K), axis=1, keepdims=True)

    hit = jnp.where(col == yl, s, 0.0)
    part = (jnp.sum(x * x, axis=(0, 1), keepdims=True)
            + jnp.sum(hit, axis=(0, 1), keepdims=True))

    @pl.when(pl.program_id(0) == 0)
    def _init():
        loss_ref[...] = jnp.zeros((1, 1), jnp.float32)

    loss_ref[...] += part


def kernel(x, y, centers):
    y2 = y.reshape(N, 1).astype(jnp.int32)
    ct = centers.T
    ynew2, loss2 = pl.pallas_call(
        _kmeans_kernel,
        grid=(GRID,),
        in_specs=[
            pl.BlockSpec((BR, D), lambda i: (i, 0)),
            pl.BlockSpec((BR, 1), lambda i: (i, 0)),
            pl.BlockSpec((D, K), lambda i: (0, 0)),
        ],
        out_specs=(
            pl.BlockSpec((BR, 1), lambda i: (i, 0)),
            pl.BlockSpec((1, 1), lambda i: (0, 0)),
        ),
        out_shape=(
            jax.ShapeDtypeStruct((N, 1), jnp.int32),
            jax.ShapeDtypeStruct((1, 1), jnp.float32),
        ),
    )(x, y2, ct)
    return (loss2[0, 0], ynew2.reshape(N))


# speedup vs baseline: 34.8886x; 1.1399x over previous
import jax
import jax.numpy as jnp
from jax.experimental import pallas as pl

N = 2048
D = 256
K = 512
BR = 2048
GRID = N // BR


def _kmeans_kernel(x_ref, y_ref, c_ref, ynew_ref, loss_ref):
    x = x_ref[...]
    ct = c_ref[...].T
    yl = y_ref[...]

    xc = jax.lax.dot_general(
        x, ct,
        dimension_numbers=(((1,), (0,)), ((), ())),
        preferred_element_type=jnp.float32,
        precision=jax.lax.Precision.HIGHEST,
    )
    c2 = jnp.sum(ct * ct, axis=0, keepdims=True)
    s = c2 - 2.0 * xc

    smin = jnp.min(s, axis=1, keepdims=True)
    col = jax.lax.broadcasted_iota(jnp.int32, (BR, K), 1)
    ynew_ref[...] = jnp.min(jnp.where(s == smin, col, K), axis=1, keepdims=True)

    hit = jnp.where(col == yl, s, 0.0)
    part = (jnp.sum(x * x, axis=(0, 1), keepdims=True)
            + jnp.sum(hit, axis=(0, 1), keepdims=True))

    @pl.when(pl.program_id(0) == 0)
    def _init():
        loss_ref[...] = jnp.zeros((1, 1), jnp.float32)

    loss_ref[...] += part


def kernel(x, y, centers):
    y2 = y.reshape(N, 1).astype(jnp.int32)
    ynew2, loss2 = pl.pallas_call(
        _kmeans_kernel,
        grid=(GRID,),
        in_specs=[
            pl.BlockSpec((BR, D), lambda i: (i, 0)),
            pl.BlockSpec((BR, 1), lambda i: (i, 0)),
            pl.BlockSpec((K, D), lambda i: (0, 0)),
        ],
        out_specs=(
            pl.BlockSpec((BR, 1), lambda i: (i, 0)),
            pl.BlockSpec((1, 1), lambda i: (0, 0)),
        ),
        out_shape=(
            jax.ShapeDtypeStruct((N, 1), jnp.int32),
            jax.ShapeDtypeStruct((1, 1), jnp.float32),
        ),
    )(x, y2, centers)
    return (loss2[0, 0], ynew2.reshape(N))


# speedup vs baseline: 55.8911x; 1.6020x over previous
import jax
import jax.numpy as jnp
from jax.experimental import pallas as pl

N = 2048
D = 256
K = 512


def _kmeans_kernel(x_ref, y_ref, c_ref, ynew_ref, loss_ref):
    x = x_ref[...]
    c = c_ref[...]
    yb = y_ref[...]

    st = jax.lax.dot_general(
        c, x,
        dimension_numbers=(((1,), (1,)), ((), ())),
        preferred_element_type=jnp.float32,
        precision=jax.lax.Precision.HIGHEST,
    )
    c2 = jnp.sum(c * c, axis=1, keepdims=True)
    s = c2 - 2.0 * st

    smin = jnp.min(s, axis=0, keepdims=True)
    row = jax.lax.broadcasted_iota(jnp.int32, (K, N), 0)
    ynew_ref[...] = jnp.min(jnp.where(s == smin, row, K), axis=0, keepdims=True)

    hit = jnp.where(row == yb, s, 0.0)
    loss_ref[...] = (jnp.sum(x * x, axis=(0, 1), keepdims=True)
                     + jnp.sum(hit, axis=(0, 1), keepdims=True))


def kernel(x, y, centers):
    y2 = y.reshape(1, N)
    ynew2, loss2 = pl.pallas_call(
        _kmeans_kernel,
        out_shape=(
            jax.ShapeDtypeStruct((1, N), jnp.int32),
            jax.ShapeDtypeStruct((1, 1), jnp.float32),
        ),
    )(x, y2, centers)
    return (loss2[0, 0], ynew2.reshape(N))


# speedup vs baseline: 72.5809x; 1.2986x over previous
import jax
import jax.numpy as jnp
from jax.experimental import pallas as pl

N = 2048
D = 256
K = 512


def _kmeans_kernel(x_ref, y_ref, c_ref, ynew_ref, loss_ref):
    x = x_ref[...]
    c = c_ref[...]
    yb = y_ref[...]

    c2x = c + c
    ch = c2x.astype(jnp.bfloat16)
    cl = (c2x - ch.astype(jnp.float32)).astype(jnp.bfloat16)
    xh = x.astype(jnp.bfloat16)
    xl = (x - xh.astype(jnp.float32)).astype(jnp.bfloat16)

    def _dot(a, b):
        return jax.lax.dot_general(
            a, b,
            dimension_numbers=(((1,), (1,)), ((), ())),
            preferred_element_type=jnp.float32,
        )

    st = _dot(ch, xh) + (_dot(ch, xl) + _dot(cl, xh))
    c2 = jnp.sum(c * c, axis=1, keepdims=True)
    s = c2 - st

    smin = jnp.min(s, axis=0, keepdims=True)
    row = jax.lax.broadcasted_iota(jnp.int32, (K, N), 0)
    ynew_ref[...] = jnp.min(jnp.where(s == smin, row, K), axis=0, keepdims=True)

    hit = jnp.where(row == yb, s, 0.0)
    loss_ref[...] = (jnp.sum(x * x, axis=(0, 1), keepdims=True)
                     + jnp.sum(hit, axis=(0, 1), keepdims=True))


def kernel(x, y, centers):
    y2 = y.reshape(1, N)
    ynew2, loss2 = pl.pallas_call(
        _kmeans_kernel,
        out_shape=(
            jax.ShapeDtypeStruct((1, N), jnp.int32),
            jax.ShapeDtypeStruct((1, 1), jnp.float32),
        ),
    )(x, y2, centers)
    return (loss2[0, 0], ynew2.reshape(N))
